# baseline (speedup 1.0000x reference)
import functools

import jax
import jax.numpy as jnp
from jax import lax
from jax.experimental import pallas as pl
from jax.experimental.pallas import tpu as pltpu
from jax.experimental.pallas import tpu_sc as plsc

N = 10000
E = 320000
D = 128
B = 100
G = 10
F1 = 32
F2 = 16

NC = 2
NS = 16
CHUNK = 128
NB = 4

EDGE_CHUNKS = (((E + CHUNK - 1) // CHUNK + NC * NS * NB - 1)
               // (NC * NS * NB)) * NC * NS * NB
EP = EDGE_CHUNKS * CHUNK

RB = 1024
NRB = (N + RB - 1) // RB

NROWS = NRB * RB


def _mesh():
    return plsc.VectorSubcoreMesh(core_axis_name="c", subcore_axis_name="s")


def _make_aggregate(n_chunks, n_rows, F):
    cpt = n_chunks // (NC * NS)
    rps = n_rows // NS

    @functools.partial(
        pl.kernel,
        out_type=jax.ShapeDtypeStruct((NC, n_rows, F), jnp.float32),
        mesh=_mesh(),
        compiler_params=pltpu.CompilerParams(use_tc_tiling_on_sc=False),
        scratch_types=[
            pltpu.VMEM((cpt, CHUNK), jnp.int32),
            pltpu.VMEM((cpt, CHUNK), jnp.int32),
        ] + [pltpu.VMEM((CHUNK, F), jnp.float32) for _ in range(NB)] + [
            pltpu.VMEM((rps, F), jnp.float32),
            pltpu.VMEM_SHARED((n_rows, F), jnp.float32),
        ] + [pltpu.SemaphoreType.DMA for _ in range(2 * NB)],
    )
    def agg(table, src3d, dst3d, zeros, out, sidx, didx, *rest):
        rows = rest[:NB]
        stage, acc = rest[NB], rest[NB + 1]
        gs = rest[NB + 2:2 * NB + 2]
        ss = rest[2 * NB + 2:]
        c = lax.axis_index("c")
        s = lax.axis_index("s")
        wid = c * NS + s
        pltpu.sync_copy(zeros.at[pl.ds(s * rps, rps)], stage)
        pltpu.sync_copy(stage, acc.at[pl.ds(s * rps, rps)])
        pltpu.sync_copy(src3d.at[wid], sidx)
        pltpu.sync_copy(dst3d.at[wid], didx)
        plsc.subcore_barrier()

        for b in range(NB):
            pltpu.async_copy(table.at[sidx.at[b]], rows[b], gs[b])

        nt = cpt // NB

        def outer(t, carry):
            g0 = t * NB
            for b in range(NB):
                j = g0 + b
                pltpu.make_async_copy(table.at[sidx.at[j]], rows[b],
                                      gs[b]).wait()
                pltpu.async_copy(rows[b], acc.at[didx.at[j]], ss[b], add=True)

            @pl.when(t < nt - 1)
            def _():
                for b in range(NB):
                    pltpu.make_async_copy(rows[b], acc.at[didx.at[g0 + b]],
                                          ss[b]).wait()
                    pltpu.async_copy(table.at[sidx.at[g0 + NB + b]], rows[b],
                                     gs[b])

            return carry

        lax.fori_loop(0, nt, outer, 0)
        for b in range(NB):
            pltpu.make_async_copy(rows[b], acc.at[didx.at[0]], ss[b]).wait()
        plsc.subcore_barrier()
        pltpu.sync_copy(acc.at[pl.ds(s * rps, rps)], stage)
        pltpu.sync_copy(stage, out.at[c, pl.ds(s * rps, rps)])

    return agg


def _make_degree(n_chunks, n_rows):
    cpt = n_chunks // (NC * NS)
    rps = n_rows // NS

    @functools.partial(
        pl.kernel,
        out_type=jax.ShapeDtypeStruct((NC * n_rows,), jnp.float32),
        mesh=_mesh(),
        compiler_params=pltpu.CompilerParams(use_tc_tiling_on_sc=False),
        scratch_types=[
            pltpu.VMEM((cpt, CHUNK), jnp.int32),
            pltpu.VMEM((CHUNK,), jnp.float32),
            pltpu.VMEM((rps,), jnp.float32),
            pltpu.VMEM_SHARED((n_rows,), jnp.float32),
            pltpu.SemaphoreType.DMA,
        ],
    )
    def deg(dst3d, zeros, out, didx, ones_v, stage, acc, sem):
        c = lax.axis_index("c")
        s = lax.axis_index("s")
        wid = c * NS + s
        for k in range(CHUNK // 16):
            ones_v[pl.ds(k * 16, 16)] = jnp.ones((16,), jnp.float32)
        pltpu.sync_copy(zeros.at[pl.ds(s * rps, rps)], stage)
        pltpu.sync_copy(stage, acc.at[pl.ds(s * rps, rps)])
        pltpu.sync_copy(dst3d.at[wid], didx)
        plsc.subcore_barrier()

        def fire(j, carry):
            pltpu.async_copy(ones_v, acc.at[didx.at[j]], sem, add=True)
            return carry

        lax.fori_loop(0, cpt, fire, 0)

        def drain(j, carry):
            pltpu.make_async_copy(ones_v, acc.at[didx.at[j]], sem).wait()
            return carry

        lax.fori_loop(0, cpt, drain, 0)
        plsc.subcore_barrier()
        pltpu.sync_copy(acc.at[pl.ds(s * rps, rps)], stage)
        pltpu.sync_copy(stage, out.at[pl.ds(c * n_rows + s * rps, rps)])

    return deg


def _stage_a(x, W1, dega, degb):
    def body(xb, w1, da, db, g1o, dvo):
        dinv = lax.rsqrt(1.0 + da[...] + db[...])
        h = jnp.dot(xb[...], w1[...], preferred_element_type=jnp.float32)
        g1o[...] = dinv[:, None] * h
        dvo[...] = dinv

    return pl.pallas_call(
        body,
        grid=(NRB,),
        in_specs=[
            pl.BlockSpec((RB, D), lambda i: (i, 0)),
            pl.BlockSpec((D, F1), lambda i: (0, 0)),
            pl.BlockSpec((RB,), lambda i: (i,)),
            pl.BlockSpec((RB,), lambda i: (i,)),
        ],
        out_specs=[
            pl.BlockSpec((RB, F1), lambda i: (i, 0)),
            pl.BlockSpec((RB,), lambda i: (i,)),
        ],
        out_shape=[
            jax.ShapeDtypeStruct((N, F1), jnp.float32),
            jax.ShapeDtypeStruct((N,), jnp.float32),
        ],
    )(x, W1, dega, degb)


def _acc_dma(acc_hbm, j, bufs, sems, slot):
    return pltpu.make_async_copy(
        acc_hbm.at[:, pl.ds(j * RB, RB), :], bufs.at[slot], sems.at[slot])


def _stage_b(acc, g1, dinv, b1, W2):
    def body(acc_hbm, g1b, dvb, b1b, w2, g2o, bufs, sems):
        i = pl.program_id(0)
        slot = lax.rem(i, 2)

        @pl.when(i == 0)
        def _():
            _acc_dma(acc_hbm, 0, bufs, sems, 0).start()

        @pl.when(i + 1 < NRB)
        def _():
            _acc_dma(acc_hbm, i + 1, bufs, sems, 1 - slot).start()

        _acc_dma(acc_hbm, i, bufs, sems, slot).wait()
        a = bufs[slot]
        dv = dvb[...][:, None]
        h = jnp.maximum(dv * (a[0] + a[1] + g1b[...]) + b1b[...], 0.0)
        g2o[...] = dv * jnp.dot(h, w2[...], preferred_element_type=jnp.float32)

    return pl.pallas_call(
        body,
        grid=(NRB,),
        in_specs=[
            pl.BlockSpec(memory_space=pl.ANY),
            pl.BlockSpec((RB, F1), lambda i: (i, 0)),
            pl.BlockSpec((RB,), lambda i: (i,)),
            pl.BlockSpec((1, F1), lambda i: (0, 0)),
            pl.BlockSpec((F1, F2), lambda i: (0, 0)),
        ],
        out_specs=pl.BlockSpec((RB, F2), lambda i: (i, 0)),
        out_shape=jax.ShapeDtypeStruct((N, F2), jnp.float32),
        scratch_shapes=[
            pltpu.VMEM((2, NC, RB, F1), jnp.float32),
            pltpu.SemaphoreType.DMA((2,)),
        ],
    )(acc, g1, dinv, b1, W2)


def _stage_e(acc, g2, dinv, b2, pa, pb, Wf, bf):
    def body(acc_hbm, g2b, dvb, b2b, pab, pbb, wfb, bfb, outb, pools, bufs,
             sems):
        i = pl.program_id(0)
        slot = lax.rem(i, 2)

        @pl.when(i == 0)
        def _():
            pools[...] = jnp.zeros((B, F2), jnp.float32)
            _acc_dma(acc_hbm, 0, bufs, sems, 0).start()

        @pl.when(i + 1 < NRB)
        def _():
            _acc_dma(acc_hbm, i + 1, bufs, sems, 1 - slot).start()

        _acc_dma(acc_hbm, i, bufs, sems, slot).wait()
        a = bufs[slot]
        h2 = jnp.maximum(
            dvb[...][:, None] * (a[0] + a[1] + g2b[...]) + b2b[...], 0.0)
        valid = (i * RB + lax.broadcasted_iota(jnp.int32, (RB, 1), 0)) < N
        h2 = jnp.where(valid, h2, 0.0)
        nid = i * RB + lax.broadcasted_iota(jnp.int32, (1, RB), 1)
        sel = ((pbb[...] <= nid) & (nid < pab[...])).astype(jnp.float32)
        pools[...] += lax.dot_general(
            sel, h2, (((1,), (0,)), ((), ())),
            preferred_element_type=jnp.float32,
            precision=lax.Precision.HIGHEST)

        @pl.when(i == NRB - 1)
        def _():
            cnt = jnp.maximum(pab[...] - pbb[...], 1).astype(jnp.float32)
            mean = pools[...] / cnt
            z = jnp.dot(mean, wfb[...],
                        preferred_element_type=jnp.float32) + bfb[...]
            m = jnp.max(z, axis=1, keepdims=True)
            outb[...] = z - (m + jnp.log(jnp.sum(jnp.exp(z - m), axis=1,
                                                 keepdims=True)))

    return pl.pallas_call(
        body,
        grid=(NRB,),
        in_specs=[
            pl.BlockSpec(memory_space=pl.ANY),
            pl.BlockSpec((RB, F2), lambda i: (i, 0)),
            pl.BlockSpec((RB,), lambda i: (i,)),
            pl.BlockSpec((1, F2), lambda i: (0, 0)),
            pl.BlockSpec((B, 1), lambda i: (0, 0)),
            pl.BlockSpec((B, 1), lambda i: (0, 0)),
            pl.BlockSpec((F2, G), lambda i: (0, 0)),
            pl.BlockSpec((1, G), lambda i: (0, 0)),
        ],
        out_specs=pl.BlockSpec((B, G), lambda i: (0, 0)),
        out_shape=jax.ShapeDtypeStruct((B, G), jnp.float32),
        scratch_shapes=[
            pltpu.VMEM((B, F2), jnp.float32),
            pltpu.VMEM((2, NC, RB, F2), jnp.float32),
            pltpu.SemaphoreType.DMA((2,)),
        ],
    )(acc, g2, dinv, b2, pa, pb, Wf, bf)


_agg_edges_f1 = _make_aggregate(EDGE_CHUNKS, NROWS, F1)
_agg_edges_f2 = _make_aggregate(EDGE_CHUNKS, NROWS, F2)
_degree = _make_degree(EDGE_CHUNKS, NROWS)


def kernel(x, edge_index, ptr, W1, b1, W2, b2, Wf, bf):
    src = edge_index[0].astype(jnp.int32)
    dst = edge_index[1].astype(jnp.int32)
    ptr = ptr.astype(jnp.int32)

    pad = EP - E
    pad_src = jnp.arange(pad, dtype=jnp.int32) % N
    pad_dst = N + (jnp.arange(pad, dtype=jnp.int32) % (NROWS - N))
    src2d = jnp.concatenate([src, pad_src]).reshape(
        NC * NS, EDGE_CHUNKS // (NC * NS), CHUNK)
    dst2d = jnp.concatenate([dst, pad_dst]).reshape(
        NC * NS, EDGE_CHUNKS // (NC * NS), CHUNK)

    zeros_deg = jnp.zeros((NROWS,), jnp.float32)
    zeros_f1 = jnp.zeros((NROWS, F1), jnp.float32)
    zeros_f2 = jnp.zeros((NROWS, F2), jnp.float32)

    degp = _degree(dst2d, zeros_deg)
    dega = degp[:N]
    degb = degp[NROWS:NROWS + N]

    g1, dinv = _stage_a(x, W1, dega, degb)

    acc1 = _agg_edges_f1(g1, src2d, dst2d, zeros_f1)
    g2 = _stage_b(acc1, g1, dinv, b1.reshape(1, F1), W2)

    acc2 = _agg_edges_f2(g2, src2d, dst2d, zeros_f2)

    pa = ptr[1:B + 1].reshape(B, 1)
    pb = ptr[0:B].reshape(B, 1)
    return _stage_e(acc2, g2, dinv, b2.reshape(1, F2),
                    pa, pb, Wf, bf.reshape(1, G))


# speedup vs baseline: 1.0399x; 1.0399x over previous
import functools

import jax
import jax.numpy as jnp
from jax import lax
from jax.experimental import pallas as pl
from jax.experimental.pallas import tpu as pltpu
from jax.experimental.pallas import tpu_sc as plsc

N = 10000
E = 320000
D = 128
B = 100
G = 10
F1 = 32
F2 = 16

NC = 2
NS = 16
CHUNK = 128
NB = 4

EDGE_CHUNKS = (((E + CHUNK - 1) // CHUNK + NC * NS * NB - 1)
               // (NC * NS * NB)) * NC * NS * NB
EP = EDGE_CHUNKS * CHUNK

RB = 2048
NRB = (N + RB - 1) // RB

NROWS = NRB * RB


def _mesh():
    return plsc.VectorSubcoreMesh(core_axis_name="c", subcore_axis_name="s")


def _make_aggregate(n_chunks, n_rows, F):
    cpt = n_chunks // (NC * NS)
    rps = n_rows // NS

    @functools.partial(
        pl.kernel,
        out_type=jax.ShapeDtypeStruct((NC, n_rows, F), jnp.float32),
        mesh=_mesh(),
        compiler_params=pltpu.CompilerParams(use_tc_tiling_on_sc=False),
        scratch_types=[
            pltpu.VMEM((cpt, CHUNK), jnp.int32),
            pltpu.VMEM((cpt, CHUNK), jnp.int32),
        ] + [pltpu.VMEM((CHUNK, F), jnp.float32) for _ in range(NB)] + [
            pltpu.VMEM((rps, F), jnp.float32),
            pltpu.VMEM_SHARED((n_rows, F), jnp.float32),
        ] + [pltpu.SemaphoreType.DMA for _ in range(2 * NB)],
    )
    def agg(table, src3d, dst3d, zeros, out, sidx, didx, *rest):
        rows = rest[:NB]
        stage, acc = rest[NB], rest[NB + 1]
        gs = rest[NB + 2:2 * NB + 2]
        ss = rest[2 * NB + 2:]
        c = lax.axis_index("c")
        s = lax.axis_index("s")
        wid = c * NS + s
        pltpu.sync_copy(zeros.at[pl.ds(s * rps, rps)], stage)
        pltpu.sync_copy(stage, acc.at[pl.ds(s * rps, rps)])
        pltpu.sync_copy(src3d.at[wid], sidx)
        pltpu.sync_copy(dst3d.at[wid], didx)
        plsc.subcore_barrier()

        for b in range(NB):
            pltpu.async_copy(table.at[sidx.at[b]], rows[b], gs[b])

        nt = cpt // NB

        def outer(t, carry):
            g0 = t * NB
            for b in range(NB):
                j = g0 + b
                pltpu.make_async_copy(table.at[sidx.at[j]], rows[b],
                                      gs[b]).wait()
                pltpu.async_copy(rows[b], acc.at[didx.at[j]], ss[b], add=True)

            @pl.when(t < nt - 1)
            def _():
                for b in range(NB):
                    pltpu.make_async_copy(rows[b], acc.at[didx.at[g0 + b]],
                                          ss[b]).wait()
                    pltpu.async_copy(table.at[sidx.at[g0 + NB + b]], rows[b],
                                     gs[b])

            return carry

        lax.fori_loop(0, nt, outer, 0)
        for b in range(NB):
            pltpu.make_async_copy(rows[b], acc.at[didx.at[0]], ss[b]).wait()
        plsc.subcore_barrier()
        pltpu.sync_copy(acc.at[pl.ds(s * rps, rps)], stage)
        pltpu.sync_copy(stage, out.at[c, pl.ds(s * rps, rps)])

    return agg


def _make_degree(n_chunks, n_rows):
    cpt = n_chunks // (NC * NS)
    rps = n_rows // NS

    @functools.partial(
        pl.kernel,
        out_type=jax.ShapeDtypeStruct((NC * n_rows,), jnp.float32),
        mesh=_mesh(),
        compiler_params=pltpu.CompilerParams(use_tc_tiling_on_sc=False),
        scratch_types=[
            pltpu.VMEM((cpt, CHUNK), jnp.int32),
            pltpu.VMEM((CHUNK,), jnp.float32),
            pltpu.VMEM((rps,), jnp.float32),
            pltpu.VMEM_SHARED((n_rows,), jnp.float32),
            pltpu.SemaphoreType.DMA,
        ],
    )
    def deg(dst3d, zeros, out, didx, ones_v, stage, acc, sem):
        c = lax.axis_index("c")
        s = lax.axis_index("s")
        wid = c * NS + s
        for k in range(CHUNK // 16):
            ones_v[pl.ds(k * 16, 16)] = jnp.ones((16,), jnp.float32)
        pltpu.sync_copy(zeros.at[pl.ds(s * rps, rps)], stage)
        pltpu.sync_copy(stage, acc.at[pl.ds(s * rps, rps)])
        pltpu.sync_copy(dst3d.at[wid], didx)
        plsc.subcore_barrier()

        def fire(j, carry):
            pltpu.async_copy(ones_v, acc.at[didx.at[j]], sem, add=True)
            return carry

        lax.fori_loop(0, cpt, fire, 0)

        def drain(j, carry):
            pltpu.make_async_copy(ones_v, acc.at[didx.at[j]], sem).wait()
            return carry

        lax.fori_loop(0, cpt, drain, 0)
        plsc.subcore_barrier()
        pltpu.sync_copy(acc.at[pl.ds(s * rps, rps)], stage)
        pltpu.sync_copy(stage, out.at[pl.ds(c * n_rows + s * rps, rps)])

    return deg


def _stage_a(x, W1, dega, degb):
    def body(xb, w1, da, db, g1o, dvo):
        dinv = lax.rsqrt(1.0 + da[...] + db[...])
        h = jnp.dot(xb[...], w1[...], preferred_element_type=jnp.float32)
        g1o[...] = dinv[:, None] * h
        dvo[...] = dinv

    return pl.pallas_call(
        body,
        grid=(NRB,),
        in_specs=[
            pl.BlockSpec((RB, D), lambda i: (i, 0)),
            pl.BlockSpec((D, F1), lambda i: (0, 0)),
            pl.BlockSpec((RB,), lambda i: (i,)),
            pl.BlockSpec((RB,), lambda i: (i,)),
        ],
        out_specs=[
            pl.BlockSpec((RB, F1), lambda i: (i, 0)),
            pl.BlockSpec((RB,), lambda i: (i,)),
        ],
        out_shape=[
            jax.ShapeDtypeStruct((N, F1), jnp.float32),
            jax.ShapeDtypeStruct((N,), jnp.float32),
        ],
    )(x, W1, dega, degb)


def _acc_dma(acc_hbm, j, bufs, sems, slot):
    return pltpu.make_async_copy(
        acc_hbm.at[:, pl.ds(j * RB, RB), :], bufs.at[slot], sems.at[slot])


def _stage_b(acc, g1, dinv, b1, W2):
    def body(acc_hbm, g1b, dvb, b1b, w2, g2o, bufs, sems):
        i = pl.program_id(0)
        slot = lax.rem(i, 2)

        @pl.when(i == 0)
        def _():
            _acc_dma(acc_hbm, 0, bufs, sems, 0).start()

        @pl.when(i + 1 < NRB)
        def _():
            _acc_dma(acc_hbm, i + 1, bufs, sems, 1 - slot).start()

        _acc_dma(acc_hbm, i, bufs, sems, slot).wait()
        a = bufs[slot]
        dv = dvb[...][:, None]
        h = jnp.maximum(dv * (a[0] + a[1] + g1b[...]) + b1b[...], 0.0)
        g2o[...] = dv * jnp.dot(h, w2[...], preferred_element_type=jnp.float32)

    return pl.pallas_call(
        body,
        grid=(NRB,),
        in_specs=[
            pl.BlockSpec(memory_space=pl.ANY),
            pl.BlockSpec((RB, F1), lambda i: (i, 0)),
            pl.BlockSpec((RB,), lambda i: (i,)),
            pl.BlockSpec((1, F1), lambda i: (0, 0)),
            pl.BlockSpec((F1, F2), lambda i: (0, 0)),
        ],
        out_specs=pl.BlockSpec((RB, F2), lambda i: (i, 0)),
        out_shape=jax.ShapeDtypeStruct((N, F2), jnp.float32),
        scratch_shapes=[
            pltpu.VMEM((2, NC, RB, F1), jnp.float32),
            pltpu.SemaphoreType.DMA((2,)),
        ],
    )(acc, g1, dinv, b1, W2)


def _stage_e(acc, g2, dinv, b2, pa, pb, Wf, bf):
    def body(acc_hbm, g2b, dvb, b2b, pab, pbb, wfb, bfb, outb, pools, bufs,
             sems):
        i = pl.program_id(0)
        slot = lax.rem(i, 2)

        @pl.when(i == 0)
        def _():
            pools[...] = jnp.zeros((B, F2), jnp.float32)
            _acc_dma(acc_hbm, 0, bufs, sems, 0).start()

        @pl.when(i + 1 < NRB)
        def _():
            _acc_dma(acc_hbm, i + 1, bufs, sems, 1 - slot).start()

        _acc_dma(acc_hbm, i, bufs, sems, slot).wait()
        a = bufs[slot]
        h2 = jnp.maximum(
            dvb[...][:, None] * (a[0] + a[1] + g2b[...]) + b2b[...], 0.0)
        valid = (i * RB + lax.broadcasted_iota(jnp.int32, (RB, 1), 0)) < N
        h2 = jnp.where(valid, h2, 0.0)
        nid = i * RB + lax.broadcasted_iota(jnp.int32, (1, RB), 1)
        sel = ((pbb[...] <= nid) & (nid < pab[...])).astype(jnp.float32)
        pools[...] += lax.dot_general(
            sel, h2, (((1,), (0,)), ((), ())),
            preferred_element_type=jnp.float32,
            precision=lax.Precision.HIGHEST)

        @pl.when(i == NRB - 1)
        def _():
            cnt = jnp.maximum(pab[...] - pbb[...], 1).astype(jnp.float32)
            mean = pools[...] / cnt
            z = jnp.dot(mean, wfb[...],
                        preferred_element_type=jnp.float32) + bfb[...]
            m = jnp.max(z, axis=1, keepdims=True)
            outb[...] = z - (m + jnp.log(jnp.sum(jnp.exp(z - m), axis=1,
                                                 keepdims=True)))

    return pl.pallas_call(
        body,
        grid=(NRB,),
        in_specs=[
            pl.BlockSpec(memory_space=pl.ANY),
            pl.BlockSpec((RB, F2), lambda i: (i, 0)),
            pl.BlockSpec((RB,), lambda i: (i,)),
            pl.BlockSpec((1, F2), lambda i: (0, 0)),
            pl.BlockSpec((B, 1), lambda i: (0, 0)),
            pl.BlockSpec((B, 1), lambda i: (0, 0)),
            pl.BlockSpec((F2, G), lambda i: (0, 0)),
            pl.BlockSpec((1, G), lambda i: (0, 0)),
        ],
        out_specs=pl.BlockSpec((B, G), lambda i: (0, 0)),
        out_shape=jax.ShapeDtypeStruct((B, G), jnp.float32),
        scratch_shapes=[
            pltpu.VMEM((B, F2), jnp.float32),
            pltpu.VMEM((2, NC, RB, F2), jnp.float32),
            pltpu.SemaphoreType.DMA((2,)),
        ],
    )(acc, g2, dinv, b2, pa, pb, Wf, bf)


_agg_edges_f1 = _make_aggregate(EDGE_CHUNKS, NROWS, F1)
_agg_edges_f2 = _make_aggregate(EDGE_CHUNKS, NROWS, F2)
_degree = _make_degree(EDGE_CHUNKS, NROWS)


def kernel(x, edge_index, ptr, W1, b1, W2, b2, Wf, bf):
    src = edge_index[0].astype(jnp.int32)
    dst = edge_index[1].astype(jnp.int32)
    ptr = ptr.astype(jnp.int32)

    pad = EP - E
    pad_src = jnp.arange(pad, dtype=jnp.int32) % N
    pad_dst = N + (jnp.arange(pad, dtype=jnp.int32) % (NROWS - N))
    src2d = jnp.concatenate([src, pad_src]).reshape(
        NC * NS, EDGE_CHUNKS // (NC * NS), CHUNK)
    dst2d = jnp.concatenate([dst, pad_dst]).reshape(
        NC * NS, EDGE_CHUNKS // (NC * NS), CHUNK)

    zeros_deg = jnp.zeros((NROWS,), jnp.float32)
    zeros_f1 = jnp.zeros((NROWS, F1), jnp.float32)
    zeros_f2 = jnp.zeros((NROWS, F2), jnp.float32)

    degp = _degree(dst2d, zeros_deg)
    dega = degp[:N]
    degb = degp[NROWS:NROWS + N]

    g1, dinv = _stage_a(x, W1, dega, degb)

    acc1 = _agg_edges_f1(g1, src2d, dst2d, zeros_f1)
    g2 = _stage_b(acc1, g1, dinv, b1.reshape(1, F1), W2)

    acc2 = _agg_edges_f2(g2, src2d, dst2d, zeros_f2)

    pa = ptr[1:B + 1].reshape(B, 1)
    pb = ptr[0:B].reshape(B, 1)
    return _stage_e(acc2, g2, dinv, b2.reshape(1, F2),
                    pa, pb, Wf, bf.reshape(1, G))


# speedup vs baseline: 1.1061x; 1.0636x over previous
import functools

import jax
import jax.numpy as jnp
from jax import lax
from jax.experimental import pallas as pl
from jax.experimental.pallas import tpu as pltpu
from jax.experimental.pallas import tpu_sc as plsc

N = 10000
E = 320000
D = 128
B = 100
G = 10
F1 = 32
F2 = 16

NC = 2
NS = 16
CHUNK = 128
NB = 8

EDGE_CHUNKS = (((E + CHUNK - 1) // CHUNK + NC * NS * NB - 1)
               // (NC * NS * NB)) * NC * NS * NB
EP = EDGE_CHUNKS * CHUNK

RB = 2048
NRB = (N + RB - 1) // RB

NROWS = NRB * RB


def _mesh():
    return plsc.VectorSubcoreMesh(core_axis_name="c", subcore_axis_name="s")


def _make_aggregate(n_chunks, n_rows, F):
    cpt = n_chunks // (NC * NS)
    rps = n_rows // NS

    @functools.partial(
        pl.kernel,
        out_type=jax.ShapeDtypeStruct((NC, n_rows, F), jnp.float32),
        mesh=_mesh(),
        compiler_params=pltpu.CompilerParams(use_tc_tiling_on_sc=False),
        scratch_types=[
            pltpu.VMEM((cpt, CHUNK), jnp.int32),
            pltpu.VMEM((cpt, CHUNK), jnp.int32),
        ] + [pltpu.VMEM((CHUNK, F), jnp.float32) for _ in range(NB)] + [
            pltpu.VMEM((rps, F), jnp.float32),
            pltpu.VMEM_SHARED((n_rows, F), jnp.float32),
        ] + [pltpu.SemaphoreType.DMA for _ in range(2 * NB)],
    )
    def agg(table, src3d, dst3d, zeros, out, sidx, didx, *rest):
        rows = rest[:NB]
        stage, acc = rest[NB], rest[NB + 1]
        gs = rest[NB + 2:2 * NB + 2]
        ss = rest[2 * NB + 2:]
        c = lax.axis_index("c")
        s = lax.axis_index("s")
        wid = c * NS + s
        pltpu.sync_copy(zeros.at[pl.ds(s * rps, rps)], stage)
        pltpu.sync_copy(stage, acc.at[pl.ds(s * rps, rps)])
        pltpu.sync_copy(src3d.at[wid], sidx)
        pltpu.sync_copy(dst3d.at[wid], didx)
        plsc.subcore_barrier()

        for b in range(NB):
            pltpu.async_copy(table.at[sidx.at[b]], rows[b], gs[b])

        nt = cpt // NB

        def outer(t, carry):
            g0 = t * NB
            for b in range(NB):
                j = g0 + b
                pltpu.make_async_copy(table.at[sidx.at[j]], rows[b],
                                      gs[b]).wait()
                pltpu.async_copy(rows[b], acc.at[didx.at[j]], ss[b], add=True)

            @pl.when(t < nt - 1)
            def _():
                for b in range(NB):
                    pltpu.make_async_copy(rows[b], acc.at[didx.at[g0 + b]],
                                          ss[b]).wait()
                    pltpu.async_copy(table.at[sidx.at[g0 + NB + b]], rows[b],
                                     gs[b])

            return carry

        lax.fori_loop(0, nt, outer, 0)
        for b in range(NB):
            pltpu.make_async_copy(rows[b], acc.at[didx.at[0]], ss[b]).wait()
        plsc.subcore_barrier()
        pltpu.sync_copy(acc.at[pl.ds(s * rps, rps)], stage)
        pltpu.sync_copy(stage, out.at[c, pl.ds(s * rps, rps)])

    return agg


def _make_degree(n_chunks, n_rows):
    cpt = n_chunks // (NC * NS)
    rps = n_rows // NS

    @functools.partial(
        pl.kernel,
        out_type=jax.ShapeDtypeStruct((NC * n_rows,), jnp.float32),
        mesh=_mesh(),
        compiler_params=pltpu.CompilerParams(use_tc_tiling_on_sc=False),
        scratch_types=[
            pltpu.VMEM((cpt, CHUNK), jnp.int32),
            pltpu.VMEM((CHUNK,), jnp.float32),
            pltpu.VMEM((rps,), jnp.float32),
            pltpu.VMEM_SHARED((n_rows,), jnp.float32),
            pltpu.SemaphoreType.DMA,
        ],
    )
    def deg(dst3d, zeros, out, didx, ones_v, stage, acc, sem):
        c = lax.axis_index("c")
        s = lax.axis_index("s")
        wid = c * NS + s
        for k in range(CHUNK // 16):
            ones_v[pl.ds(k * 16, 16)] = jnp.ones((16,), jnp.float32)
        pltpu.sync_copy(zeros.at[pl.ds(s * rps, rps)], stage)
        pltpu.sync_copy(stage, acc.at[pl.ds(s * rps, rps)])
        pltpu.sync_copy(dst3d.at[wid], didx)
        plsc.subcore_barrier()

        def fire(j, carry):
            pltpu.async_copy(ones_v, acc.at[didx.at[j]], sem, add=True)
            return carry

        lax.fori_loop(0, cpt, fire, 0)

        def drain(j, carry):
            pltpu.make_async_copy(ones_v, acc.at[didx.at[j]], sem).wait()
            return carry

        lax.fori_loop(0, cpt, drain, 0)
        plsc.subcore_barrier()
        pltpu.sync_copy(acc.at[pl.ds(s * rps, rps)], stage)
        pltpu.sync_copy(stage, out.at[pl.ds(c * n_rows + s * rps, rps)])

    return deg


def _stage_a(x, W1, dega, degb):
    def body(xb, w1, da, db, g1o, dvo):
        dinv = lax.rsqrt(1.0 + da[...] + db[...])
        h = jnp.dot(xb[...], w1[...], preferred_element_type=jnp.float32)
        g1o[...] = dinv[:, None] * h
        dvo[...] = dinv

    return pl.pallas_call(
        body,
        grid=(NRB,),
        in_specs=[
            pl.BlockSpec((RB, D), lambda i: (i, 0)),
            pl.BlockSpec((D, F1), lambda i: (0, 0)),
            pl.BlockSpec((RB,), lambda i: (i,)),
            pl.BlockSpec((RB,), lambda i: (i,)),
        ],
        out_specs=[
            pl.BlockSpec((RB, F1), lambda i: (i, 0)),
            pl.BlockSpec((RB,), lambda i: (i,)),
        ],
        out_shape=[
            jax.ShapeDtypeStruct((N, F1), jnp.float32),
            jax.ShapeDtypeStruct((N,), jnp.float32),
        ],
    )(x, W1, dega, degb)


def _acc_dma(acc_hbm, j, bufs, sems, slot):
    return pltpu.make_async_copy(
        acc_hbm.at[:, pl.ds(j * RB, RB), :], bufs.at[slot], sems.at[slot])


def _stage_b(acc, g1, dinv, b1, W2):
    def body(acc_hbm, g1b, dvb, b1b, w2, g2o, bufs, sems):
        i = pl.program_id(0)
        slot = lax.rem(i, 2)

        @pl.when(i == 0)
        def _():
            _acc_dma(acc_hbm, 0, bufs, sems, 0).start()

        @pl.when(i + 1 < NRB)
        def _():
            _acc_dma(acc_hbm, i + 1, bufs, sems, 1 - slot).start()

        _acc_dma(acc_hbm, i, bufs, sems, slot).wait()
        a = bufs[slot]
        dv = dvb[...][:, None]
        h = jnp.maximum(dv * (a[0] + a[1] + g1b[...]) + b1b[...], 0.0)
        g2o[...] = dv * jnp.dot(h, w2[...], preferred_element_type=jnp.float32)

    return pl.pallas_call(
        body,
        grid=(NRB,),
        in_specs=[
            pl.BlockSpec(memory_space=pl.ANY),
            pl.BlockSpec((RB, F1), lambda i: (i, 0)),
            pl.BlockSpec((RB,), lambda i: (i,)),
            pl.BlockSpec((1, F1), lambda i: (0, 0)),
            pl.BlockSpec((F1, F2), lambda i: (0, 0)),
        ],
        out_specs=pl.BlockSpec((RB, F2), lambda i: (i, 0)),
        out_shape=jax.ShapeDtypeStruct((N, F2), jnp.float32),
        scratch_shapes=[
            pltpu.VMEM((2, NC, RB, F1), jnp.float32),
            pltpu.SemaphoreType.DMA((2,)),
        ],
    )(acc, g1, dinv, b1, W2)


def _stage_e(acc, g2, dinv, b2, pa, pb, Wf, bf):
    def body(acc_hbm, g2b, dvb, b2b, pab, pbb, wfb, bfb, outb, pools, bufs,
             sems):
        i = pl.program_id(0)
        slot = lax.rem(i, 2)

        @pl.when(i == 0)
        def _():
            pools[...] = jnp.zeros((B, F2), jnp.float32)
            _acc_dma(acc_hbm, 0, bufs, sems, 0).start()

        @pl.when(i + 1 < NRB)
        def _():
            _acc_dma(acc_hbm, i + 1, bufs, sems, 1 - slot).start()

        _acc_dma(acc_hbm, i, bufs, sems, slot).wait()
        a = bufs[slot]
        h2 = jnp.maximum(
            dvb[...][:, None] * (a[0] + a[1] + g2b[...]) + b2b[...], 0.0)
        valid = (i * RB + lax.broadcasted_iota(jnp.int32, (RB, 1), 0)) < N
        h2 = jnp.where(valid, h2, 0.0)
        nid = i * RB + lax.broadcasted_iota(jnp.int32, (1, RB), 1)
        sel = ((pbb[...] <= nid) & (nid < pab[...])).astype(jnp.float32)
        pools[...] += lax.dot_general(
            sel, h2, (((1,), (0,)), ((), ())),
            preferred_element_type=jnp.float32,
            precision=lax.Precision.HIGHEST)

        @pl.when(i == NRB - 1)
        def _():
            cnt = jnp.maximum(pab[...] - pbb[...], 1).astype(jnp.float32)
            mean = pools[...] / cnt
            z = jnp.dot(mean, wfb[...],
                        preferred_element_type=jnp.float32) + bfb[...]
            m = jnp.max(z, axis=1, keepdims=True)
            outb[...] = z - (m + jnp.log(jnp.sum(jnp.exp(z - m), axis=1,
                                                 keepdims=True)))

    return pl.pallas_call(
        body,
        grid=(NRB,),
        in_specs=[
            pl.BlockSpec(memory_space=pl.ANY),
            pl.BlockSpec((RB, F2), lambda i: (i, 0)),
            pl.BlockSpec((RB,), lambda i: (i,)),
            pl.BlockSpec((1, F2), lambda i: (0, 0)),
            pl.BlockSpec((B, 1), lambda i: (0, 0)),
            pl.BlockSpec((B, 1), lambda i: (0, 0)),
            pl.BlockSpec((F2, G), lambda i: (0, 0)),
            pl.BlockSpec((1, G), lambda i: (0, 0)),
        ],
        out_specs=pl.BlockSpec((B, G), lambda i: (0, 0)),
        out_shape=jax.ShapeDtypeStruct((B, G), jnp.float32),
        scratch_shapes=[
            pltpu.VMEM((B, F2), jnp.float32),
            pltpu.VMEM((2, NC, RB, F2), jnp.float32),
            pltpu.SemaphoreType.DMA((2,)),
        ],
    )(acc, g2, dinv, b2, pa, pb, Wf, bf)


_agg_edges_f1 = _make_aggregate(EDGE_CHUNKS, NROWS, F1)
_agg_edges_f2 = _make_aggregate(EDGE_CHUNKS, NROWS, F2)
_degree = _make_degree(EDGE_CHUNKS, NROWS)


def kernel(x, edge_index, ptr, W1, b1, W2, b2, Wf, bf):
    src = edge_index[0].astype(jnp.int32)
    dst = edge_index[1].astype(jnp.int32)
    ptr = ptr.astype(jnp.int32)

    pad = EP - E
    pad_src = jnp.arange(pad, dtype=jnp.int32) % N
    pad_dst = N + (jnp.arange(pad, dtype=jnp.int32) % (NROWS - N))
    src2d = jnp.concatenate([src, pad_src]).reshape(
        NC * NS, EDGE_CHUNKS // (NC * NS), CHUNK)
    dst2d = jnp.concatenate([dst, pad_dst]).reshape(
        NC * NS, EDGE_CHUNKS // (NC * NS), CHUNK)

    zeros_deg = jnp.zeros((NROWS,), jnp.float32)
    zeros_f1 = jnp.zeros((NROWS, F1), jnp.float32)
    zeros_f2 = jnp.zeros((NROWS, F2), jnp.float32)

    degp = _degree(dst2d, zeros_deg)
    dega = degp[:N]
    degb = degp[NROWS:NROWS + N]

    g1, dinv = _stage_a(x, W1, dega, degb)

    acc1 = _agg_edges_f1(g1, src2d, dst2d, zeros_f1)
    g2 = _stage_b(acc1, g1, dinv, b1.reshape(1, F1), W2)

    acc2 = _agg_edges_f2(g2, src2d, dst2d, zeros_f2)

    pa = ptr[1:B + 1].reshape(B, 1)
    pb = ptr[0:B].reshape(B, 1)
    return _stage_e(acc2, g2, dinv, b2.reshape(1, F2),
                    pa, pb, Wf, bf.reshape(1, G))
